# baseline (speedup 1.0000x reference)
import functools

import jax
import jax.numpy as jnp
from jax.experimental import pallas as pl
from jax.experimental.pallas import tpu as pltpu
from jax.experimental.pallas import tpu_sc as plsc

_HID = 128
_ROWS = 1024
_GATHER_WINDOW = 128
_WINDOWS_PER_STEP = 2
_NSUB = 16


def _project_body(table_ref, w_ref, p_ref):
    nrows = table_ref.shape[0]
    p_ref[:nrows, :] = jnp.dot(table_ref[...], w_ref[...],
                               preferred_element_type=jnp.float32)


def _project(table, W):
    return pl.pallas_call(
        _project_body,
        out_shape=jax.ShapeDtypeStruct((_ROWS, W.shape[1]), jnp.float32),
    )(table, W)


def _gather(p, idx):
    n = idx.shape[0]
    w, k_w = _GATHER_WINDOW, _WINDOWS_PER_STEP
    idx2 = idx.reshape(n // w, w)
    mesh = plsc.VectorSubcoreMesh(core_axis_name="c", subcore_axis_name="s")
    rows_per_sub = _ROWS // _NSUB

    @functools.partial(
        pl.kernel,
        out_type=jax.ShapeDtypeStruct((n, _HID), jnp.float32),
        mesh=mesh,
        scratch_types=[pltpu.VMEM_SHARED((_ROWS, _HID), jnp.float32)]
        + [pltpu.SemaphoreType.DMA] * k_w,
    )
    def k(p_hbm, i_hbm, o_hbm, p_shared, *sems):
        sid = jax.lax.axis_index("s")
        sl = pl.ds(sid * rows_per_sub, rows_per_sub)
        pltpu.sync_copy(p_hbm.at[sl], p_shared.at[sl])
        plsc.subcore_barrier()

        def body(i_vmem, o_vmem):
            copies = [
                pltpu.async_copy(p_shared.at[i_vmem.at[j]],
                                 o_vmem.at[pl.ds(j * w, w)], sems[j])
                for j in range(k_w)
            ]
            for c in copies:
                c.wait()

        pltpu.emit_pipeline(
            body,
            grid=(n // (w * k_w),),
            in_specs=[pl.BlockSpec((k_w, w), lambda i: (i, 0))],
            out_specs=[pl.BlockSpec((k_w * w, _HID), lambda i: (i, 0))],
            core_axis_name=("c", "s"),
            dimension_semantics=(pltpu.PARALLEL,),
        )(i_hbm, o_hbm)

    return k(p, idx2)


def kernel(X, table, W):
    flat = X.reshape(-1).astype(jnp.int32)
    p = _project(table, W)
    return _gather(p, flat)


# speedup vs baseline: 1.0043x; 1.0043x over previous
import functools

import jax
import jax.numpy as jnp
from jax.experimental import pallas as pl
from jax.experimental.pallas import tpu as pltpu
from jax.experimental.pallas import tpu_sc as plsc

_HID = 128
_ROWS = 1024
_GATHER_WINDOW = 64
_WINDOWS_PER_STEP = 4
_NSUB = 16


def _project_body(table_ref, w_ref, p_ref):
    nrows = table_ref.shape[0]
    p_ref[:nrows, :] = jnp.dot(table_ref[...], w_ref[...],
                               preferred_element_type=jnp.float32)


def _project(table, W):
    return pl.pallas_call(
        _project_body,
        out_shape=jax.ShapeDtypeStruct((_ROWS, W.shape[1]), jnp.float32),
    )(table, W)


def _gather(p, idx):
    n = idx.shape[0]
    w, k_w = _GATHER_WINDOW, _WINDOWS_PER_STEP
    idx2 = idx.reshape(n // w, w)
    mesh = plsc.VectorSubcoreMesh(core_axis_name="c", subcore_axis_name="s")
    rows_per_sub = _ROWS // _NSUB

    @functools.partial(
        pl.kernel,
        out_type=jax.ShapeDtypeStruct((n, _HID), jnp.float32),
        mesh=mesh,
        scratch_types=[pltpu.VMEM_SHARED((_ROWS, _HID), jnp.float32)]
        + [pltpu.SemaphoreType.DMA] * k_w,
    )
    def k(p_hbm, i_hbm, o_hbm, p_shared, *sems):
        sid = jax.lax.axis_index("s")
        sl = pl.ds(sid * rows_per_sub, rows_per_sub)
        pltpu.sync_copy(p_hbm.at[sl], p_shared.at[sl])
        plsc.subcore_barrier()

        def body(i_vmem, o_vmem):
            copies = [
                pltpu.async_copy(p_shared.at[i_vmem.at[j]],
                                 o_vmem.at[pl.ds(j * w, w)], sems[j])
                for j in range(k_w)
            ]
            for c in copies:
                c.wait()

        pltpu.emit_pipeline(
            body,
            grid=(n // (w * k_w),),
            in_specs=[pl.BlockSpec((k_w, w), lambda i: (i, 0))],
            out_specs=[pl.BlockSpec((k_w * w, _HID), lambda i: (i, 0))],
            core_axis_name=("c", "s"),
            dimension_semantics=(pltpu.PARALLEL,),
        )(i_hbm, o_hbm)

    return k(p, idx2)


def kernel(X, table, W):
    flat = X.reshape(-1).astype(jnp.int32)
    p = _project(table, W)
    return _gather(p, flat)
